# baseline (speedup 1.0000x reference)
import jax
import jax.numpy as jnp
from jax import lax
from jax.experimental import pallas as pl
from jax.experimental.pallas import tpu as pltpu
from jax.experimental.pallas import tpu_sc as plsc

_B = 4096
_D = 64
_NNEG = 64
_DECAY = 1e-4
_T2 = 1.05
_NC = 2
_NS = 16
_NW = _NC * _NS
_BPW = _B // _NW
_L = 16

_PARAMS = pltpu.CompilerParams(needs_layout_passes=False,
                               use_tc_tiling_on_sc=False,
                               disable_bounds_checks=True)


def _wid_base():
    wid = lax.axis_index("s") * _NC + lax.axis_index("c")
    return wid * _BPW


def _sci_body(p_idx_hbm, n_idx_hbm, itab_hbm, usel_hbm,
              py_hbm, nsum_hbm, reg_hbm,
              p_idx_v, n_idx_v, u_rows, p_rows,
              it_buf0, it_buf1, it_buf2, it_buf3,
              py_loc, nsum_loc, reg_loc, u_pad,
              sem_u, sem_p, sem0, sem1, sem2, sem3):
    base = _wid_base()

    pltpu.sync_copy(p_idx_hbm.at[pl.ds(base, _BPW)], p_idx_v)
    pltpu.sync_copy(
        n_idx_hbm.at[pl.ds(pl.multiple_of(base * _NNEG, 8), _BPW * _NNEG)],
        n_idx_v)
    cp_u = pltpu.async_copy(
        usel_hbm.at[pl.ds(pl.multiple_of(base, 8), _BPW), :], u_rows, sem_u)
    cp_p = pltpu.async_copy(itab_hbm.at[p_idx_v], p_rows, sem_p)

    bufs = (it_buf0, it_buf1, it_buf2, it_buf3)
    sems = (sem0, sem1, sem2, sem3)

    def idx_pair(q):
        return n_idx_v.at[pl.ds(pl.multiple_of(q * 2 * _NNEG, 8),
                                2 * _NNEG)]

    def issue(q, s):
        pltpu.async_copy(itab_hbm.at[idx_pair(q)], bufs[s], sems[s])

    def wait(q, s):
        pltpu.make_async_copy(itab_hbm.at[idx_pair(q)], bufs[s],
                              sems[s]).wait()

    for s in range(4):
        issue(s, s)
    cp_u.wait()
    cp_p.wait()

    lanes = lax.iota(jnp.int32, _L)
    zero16 = jnp.zeros((_L,), jnp.float32)

    def rsqrt16(x):
        x = jnp.maximum(x, 1e-24)
        i = plsc.bitcast(x, jnp.int32)
        i = 0x5F3759DF - lax.shift_right_logical(i, 1)
        y = plsc.bitcast(i, jnp.float32)
        for _ in range(3):
            y = y * (1.5 - 0.5 * x * y * y)
        return y

    def col_step(colw, d_next):
        colw = colw + 1
        if d_next >= _D - _L + 1:
            colw = colw + jnp.where(lanes == _D - d_next, -_D, 0)
        return colw

    def up_body(bg, carry):
        a_su = zero16
        a_pd = zero16
        a_ps = zero16
        rows = bg * _L + lanes
        colw = lanes
        for d in range(_D):
            vu = plsc.load_gather(u_rows, [rows, colw])
            vp = plsc.load_gather(p_rows, [rows, colw])
            a_su = a_su + vu * vu
            a_pd = a_pd + vu * vp
            a_ps = a_ps + vp * vp
            colw = col_step(colw, d + 1)
        py_loc[pl.ds(bg * _L, _L)] = a_pd * rsqrt16(a_ps) * rsqrt16(a_su)
        reg_loc[pl.ds(bg * _L, _L)] = a_su + a_ps
        nsum_loc[pl.ds(bg * _L, _L)] = zero16
        return carry

    lax.fori_loop(0, _BPW // _L, up_body, 0)

    _OFFS = (0, _L, 2 * _L, 3 * _L)

    def compute_b(b, buf, half):
        for k in range(_D // _L):
            u_pad[pl.ds(k * _L, _L)] = u_rows[b, pl.ds(k * _L, _L)]
        u_pad[pl.ds(_D, _L)] = u_rows[b, pl.ds(0, _L)]
        row_ids = [half * _NNEG + off + lanes for off in _OFFS]

        def dblock(blk, carry):
            colw = carry[0]
            dot_acc = list(carry[1:5])
            si_acc = list(carry[5:9])
            d0 = blk * 8
            for j in range(8):
                uw = u_pad[pl.ds(d0 + j, _L)]
                for g in range(4):
                    v = plsc.load_gather(buf, [row_ids[g], colw])
                    dot_acc[g] = dot_acc[g] + v * uw
                    si_acc[g] = si_acc[g] + v * v
                colw = colw + 1 + jnp.where(lanes == _D - (d0 + j + 1),
                                            -_D, 0)
            return (colw, *dot_acc, *si_acc)

        res = lax.fori_loop(0, _D // 8, dblock, (lanes, *([zero16] * 8)))
        dot_acc = list(res[1:5])
        si_acc = list(res[5:9])
        uv = [u_pad[pl.ds(k * _L, _L)] for k in range(_D // _L)]
        su_b = jnp.sum(uv[0] * uv[0] + uv[1] * uv[1]
                       + uv[2] * uv[2] + uv[3] * uv[3])
        ru = rsqrt16(jnp.full((_L,), su_b, jnp.float32))
        e_acc = zero16
        s_acc = zero16
        for g in range(4):
            y = dot_acc[g] * rsqrt16(si_acc[g]) * ru
            e_acc = e_acc + jnp.exp(y * 10.0)
            s_acc = s_acc + si_acc[g]
        ns_b = jnp.sum(e_acc)
        rs_b = jnp.sum(s_acc)
        blk = (b // _L) * _L
        lane_mask = lanes == lax.rem(b, _L)
        plsc.addupdate(nsum_loc.at[pl.ds(blk, _L)],
                       jnp.where(lane_mask, jnp.full((_L,), ns_b), zero16))
        plsc.addupdate(reg_loc.at[pl.ds(blk, _L)],
                       jnp.where(lane_mask, jnp.full((_L,), rs_b), zero16))

    def compute_q(q, s):
        wait(q, s)
        compute_b(2 * q, bufs[s], 0)
        compute_b(2 * q + 1, bufs[s], 1)
        issue(lax.rem(q + 4, _BPW // 2), s)

    def outer(i, carry):
        for s in range(4):
            compute_q(4 * i + s, s)
        return carry

    lax.fori_loop(0, _BPW // 8, outer, 0)

    for s in range(4):
        wait(s, s)

    pltpu.sync_copy(py_loc, py_hbm.at[pl.ds(base, _BPW)])
    pltpu.sync_copy(nsum_loc, nsum_hbm.at[pl.ds(base, _BPW)])
    pltpu.sync_copy(reg_loc, reg_hbm.at[pl.ds(base, _BPW)])


def _sci_call(pos_i, negs, item_embed, u_sel):
    mesh = plsc.VectorSubcoreMesh(core_axis_name="c", subcore_axis_name="s",
                                  num_cores=_NC, num_subcores=_NS)
    f = pl.kernel(
        _sci_body,
        out_type=[
            jax.ShapeDtypeStruct((_B,), jnp.float32),
            jax.ShapeDtypeStruct((_B,), jnp.float32),
            jax.ShapeDtypeStruct((_B,), jnp.float32),
        ],
        mesh=mesh,
        scratch_types=[
            pltpu.VMEM((_BPW,), jnp.int32),
            pltpu.VMEM((_BPW * _NNEG,), jnp.int32),
            pltpu.VMEM((_BPW, _D), jnp.float32),
            pltpu.VMEM((_BPW, _D), jnp.float32),
            pltpu.VMEM((2 * _NNEG, _D), jnp.float32),
            pltpu.VMEM((2 * _NNEG, _D), jnp.float32),
            pltpu.VMEM((2 * _NNEG, _D), jnp.float32),
            pltpu.VMEM((2 * _NNEG, _D), jnp.float32),
            pltpu.VMEM((_BPW,), jnp.float32),
            pltpu.VMEM((_BPW,), jnp.float32),
            pltpu.VMEM((_BPW,), jnp.float32),
            pltpu.VMEM((_D + _L,), jnp.float32),
            pltpu.SemaphoreType.DMA,
            pltpu.SemaphoreType.DMA,
            pltpu.SemaphoreType.DMA,
            pltpu.SemaphoreType.DMA,
            pltpu.SemaphoreType.DMA,
            pltpu.SemaphoreType.DMA,
        ],
        compiler_params=_PARAMS,
    )
    return f(pos_i, negs, item_embed, u_sel)


def _tc_body(py_ref, nsum_ref, reg_ref, loss_ref, emb_ref):
    py = py_ref[...]
    nsum = nsum_ref[...]
    regp = reg_ref[...]
    logsum = jnp.sum(jnp.log(nsum))
    loss = -(10.0 * jnp.sum(py) - _T2 * logsum) / _B
    reg = 0.5 * jnp.sum(regp)
    emb = _DECAY * reg / _B
    loss_ref[0, 0] = loss + emb
    emb_ref[0, 0] = emb


def _tc_call(py, nsum, regp):
    return pl.pallas_call(
        _tc_body,
        out_shape=[
            jax.ShapeDtypeStruct((1, 1), jnp.float32),
            jax.ShapeDtypeStruct((1, 1), jnp.float32),
        ],
        out_specs=[
            pl.BlockSpec(memory_space=pltpu.SMEM),
            pl.BlockSpec(memory_space=pltpu.SMEM),
        ],
    )(py, nsum, regp)


def kernel(user_embed, item_embed, users, pos_items, neg_items):
    users_i = users.astype(jnp.int32)
    pos_i = pos_items.astype(jnp.int32)
    negs = neg_items.astype(jnp.int32).reshape(-1)
    u_sel = user_embed.at[users_i].get(mode="promise_in_bounds")
    py, nsum, regp = _sci_call(pos_i, negs, item_embed, u_sel)
    loss2, emb2 = _tc_call(py.reshape(32, 128), nsum.reshape(32, 128),
                           regp.reshape(32, 128))
    return (loss2[0, 0], emb2[0, 0])


# speedup vs baseline: 1.0757x; 1.0757x over previous
import jax
import jax.numpy as jnp
from jax import lax
from jax.experimental import pallas as pl
from jax.experimental.pallas import tpu as pltpu
from jax.experimental.pallas import tpu_sc as plsc

_B = 4096
_D = 64
_NNEG = 64
_DECAY = 1e-4
_T2 = 1.05
_NC = 2
_NS = 16
_NW = _NC * _NS
_BPW = _B // _NW
_L = 16

_PARAMS = pltpu.CompilerParams(needs_layout_passes=False,
                               use_tc_tiling_on_sc=False,
                               disable_bounds_checks=True)


def _wid_base():
    wid = lax.axis_index("s") * _NC + lax.axis_index("c")
    return wid * _BPW


def _sci_body(p_idx_hbm, n_idx_hbm, itab_hbm, usel_hbm,
              nd_hbm, ns_hbm, pd_hbm, ps_hbm, su_hbm,
              p_idx_v, n_idx_v, u_rows, p_rows,
              it_buf0, it_buf1, it_buf2, it_buf3,
              nd_loc, ns_loc, pd_loc, ps_loc, su_loc, u_pad,
              sem_u, sem_p, sem0, sem1, sem2, sem3):
    base = _wid_base()

    pltpu.sync_copy(p_idx_hbm.at[pl.ds(base, _BPW)], p_idx_v)
    pltpu.sync_copy(
        n_idx_hbm.at[pl.ds(pl.multiple_of(base * _NNEG, 8), _BPW * _NNEG)],
        n_idx_v)
    cp_u = pltpu.async_copy(
        usel_hbm.at[pl.ds(pl.multiple_of(base, 8), _BPW), :], u_rows, sem_u)
    cp_p = pltpu.async_copy(itab_hbm.at[p_idx_v], p_rows, sem_p)

    bufs = (it_buf0, it_buf1, it_buf2, it_buf3)
    sems = (sem0, sem1, sem2, sem3)

    def idx_pair(q):
        return n_idx_v.at[pl.ds(pl.multiple_of(q * 2 * _NNEG, 8),
                                2 * _NNEG)]

    def issue(q, s):
        pltpu.async_copy(itab_hbm.at[idx_pair(q)], bufs[s], sems[s])

    def wait(q, s):
        pltpu.make_async_copy(itab_hbm.at[idx_pair(q)], bufs[s],
                              sems[s]).wait()

    for s in range(4):
        issue(s, s)
    cp_u.wait()
    cp_p.wait()

    lanes = lax.iota(jnp.int32, _L)
    zero16 = jnp.zeros((_L,), jnp.float32)

    def col_step(colw, d_next):
        colw = colw + 1
        if d_next >= _D - _L + 1:
            colw = colw + jnp.where(lanes == _D - d_next, -_D, 0)
        return colw

    def up_body(bg, carry):
        a_su = zero16
        a_pd = zero16
        a_ps = zero16
        rows = bg * _L + lanes
        colw = lanes
        for d in range(_D):
            vu = plsc.load_gather(u_rows, [rows, colw])
            vp = plsc.load_gather(p_rows, [rows, colw])
            a_su = a_su + vu * vu
            a_pd = a_pd + vu * vp
            a_ps = a_ps + vp * vp
            colw = col_step(colw, d + 1)
        su_loc[pl.ds(bg * _L, _L)] = a_su
        pd_loc[pl.ds(bg * _L, _L)] = a_pd
        ps_loc[pl.ds(bg * _L, _L)] = a_ps
        return carry

    lax.fori_loop(0, _BPW // _L, up_body, 0)

    _OFFS = (0, _L, 2 * _L, 3 * _L)

    def compute_b(b, buf, half):
        for k in range(_D // _L):
            u_pad[pl.ds(k * _L, _L)] = u_rows[b, pl.ds(k * _L, _L)]
        u_pad[pl.ds(_D, _L)] = u_rows[b, pl.ds(0, _L)]
        row_ids = [half * _NNEG + off + lanes for off in _OFFS]

        def dblock(blk, carry):
            colw = carry[0]
            dot_acc = list(carry[1:5])
            si_acc = list(carry[5:9])
            d0 = blk * 8
            for j in range(8):
                uw = u_pad[pl.ds(d0 + j, _L)]
                for g in range(4):
                    v = plsc.load_gather(buf, [row_ids[g], colw])
                    dot_acc[g] = dot_acc[g] + v * uw
                    si_acc[g] = si_acc[g] + v * v
                colw = colw + 1 + jnp.where(lanes == _D - (d0 + j + 1),
                                            -_D, 0)
            return (colw, *dot_acc, *si_acc)

        res = lax.fori_loop(0, _D // 8, dblock, (lanes, *([zero16] * 8)))
        dot_acc = list(res[1:5])
        si_acc = list(res[5:9])
        for g in range(4):
            nd_loc[pl.ds(b * _NNEG + _OFFS[g], _L)] = dot_acc[g]
            ns_loc[pl.ds(b * _NNEG + _OFFS[g], _L)] = si_acc[g]

    def compute_q(q, s):
        wait(q, s)
        compute_b(2 * q, bufs[s], 0)
        compute_b(2 * q + 1, bufs[s], 1)
        issue(lax.rem(q + 4, _BPW // 2), s)

    def outer(i, carry):
        for s in range(4):
            compute_q(4 * i + s, s)
        return carry

    lax.fori_loop(0, _BPW // 8, outer, 0)

    for s in range(4):
        wait(s, s)

    out_off = pl.multiple_of(base * _NNEG, 8)
    pltpu.sync_copy(nd_loc, nd_hbm.at[pl.ds(out_off, _BPW * _NNEG)])
    pltpu.sync_copy(ns_loc, ns_hbm.at[pl.ds(out_off, _BPW * _NNEG)])
    pltpu.sync_copy(pd_loc, pd_hbm.at[pl.ds(base, _BPW)])
    pltpu.sync_copy(ps_loc, ps_hbm.at[pl.ds(base, _BPW)])
    pltpu.sync_copy(su_loc, su_hbm.at[pl.ds(base, _BPW)])


def _sci_call(pos_i, negs, item_embed, u_sel):
    mesh = plsc.VectorSubcoreMesh(core_axis_name="c", subcore_axis_name="s",
                                  num_cores=_NC, num_subcores=_NS)
    f = pl.kernel(
        _sci_body,
        out_type=[
            jax.ShapeDtypeStruct((_B * _NNEG,), jnp.float32),
            jax.ShapeDtypeStruct((_B * _NNEG,), jnp.float32),
            jax.ShapeDtypeStruct((_B,), jnp.float32),
            jax.ShapeDtypeStruct((_B,), jnp.float32),
            jax.ShapeDtypeStruct((_B,), jnp.float32),
        ],
        mesh=mesh,
        scratch_types=[
            pltpu.VMEM((_BPW,), jnp.int32),
            pltpu.VMEM((_BPW * _NNEG,), jnp.int32),
            pltpu.VMEM((_BPW, _D), jnp.float32),
            pltpu.VMEM((_BPW, _D), jnp.float32),
            pltpu.VMEM((2 * _NNEG, _D), jnp.float32),
            pltpu.VMEM((2 * _NNEG, _D), jnp.float32),
            pltpu.VMEM((2 * _NNEG, _D), jnp.float32),
            pltpu.VMEM((2 * _NNEG, _D), jnp.float32),
            pltpu.VMEM((_BPW * _NNEG,), jnp.float32),
            pltpu.VMEM((_BPW * _NNEG,), jnp.float32),
            pltpu.VMEM((_BPW,), jnp.float32),
            pltpu.VMEM((_BPW,), jnp.float32),
            pltpu.VMEM((_BPW,), jnp.float32),
            pltpu.VMEM((_D + _L,), jnp.float32),
            pltpu.SemaphoreType.DMA,
            pltpu.SemaphoreType.DMA,
            pltpu.SemaphoreType.DMA,
            pltpu.SemaphoreType.DMA,
            pltpu.SemaphoreType.DMA,
            pltpu.SemaphoreType.DMA,
        ],
        compiler_params=_PARAMS,
    )
    return f(pos_i, negs, item_embed, u_sel)


def _tc_body(nd_ref, ns_ref, pd_ref, ps_ref, su_ref, su2_ref,
             loss_ref, emb_ref):
    nd = nd_ref[...]
    ns = ns_ref[...]
    pd = pd_ref[...]
    ps = ps_ref[...]
    su = su_ref[...]
    su2 = su2_ref[...]
    nu2 = jnp.maximum(jnp.sqrt(su2), 1e-12)
    yn = nd / jnp.maximum(jnp.sqrt(ns), 1e-12)
    y_l = yn[:, :_NNEG] / nu2[:, 0:1]
    y_r = yn[:, _NNEG:] / nu2[:, 1:2]
    ls_l = jnp.log(jnp.sum(jnp.exp(y_l * 10.0), axis=1, keepdims=True))
    ls_r = jnp.log(jnp.sum(jnp.exp(y_r * 10.0), axis=1, keepdims=True))
    logsum = jnp.sum(ls_l) + jnp.sum(ls_r)
    y_pos = pd / (jnp.maximum(jnp.sqrt(ps), 1e-12)
                  * jnp.maximum(jnp.sqrt(su), 1e-12))
    loss = -(10.0 * jnp.sum(y_pos) - _T2 * logsum) / _B
    reg = 0.5 * (jnp.sum(su) + jnp.sum(ps) + jnp.sum(ns))
    emb = _DECAY * reg / _B
    loss_ref[0, 0] = loss + emb
    emb_ref[0, 0] = emb


def _tc_call(nd, ns, pd, ps, su, su2):
    return pl.pallas_call(
        _tc_body,
        out_shape=[
            jax.ShapeDtypeStruct((1, 1), jnp.float32),
            jax.ShapeDtypeStruct((1, 1), jnp.float32),
        ],
        out_specs=[
            pl.BlockSpec(memory_space=pltpu.SMEM),
            pl.BlockSpec(memory_space=pltpu.SMEM),
        ],
    )(nd, ns, pd, ps, su, su2)


def kernel(user_embed, item_embed, users, pos_items, neg_items):
    users_i = users.astype(jnp.int32)
    pos_i = pos_items.astype(jnp.int32)
    negs = neg_items.astype(jnp.int32).reshape(-1)
    u_sel = user_embed.at[users_i].get(mode="promise_in_bounds")
    nd, ns, pd, ps, su = _sci_call(pos_i, negs, item_embed, u_sel)
    h = _B // 2
    loss2, emb2 = _tc_call(nd.reshape(h, 128), ns.reshape(h, 128),
                           pd.reshape(32, 128), ps.reshape(32, 128),
                           su.reshape(32, 128), su.reshape(h, 2))
    return (loss2[0, 0], emb2[0, 0])
